# baseline (speedup 1.0000x reference)
import jax
import jax.numpy as jnp
from jax.experimental import pallas as pl

_B = 64
_N = 128
_P = _N // 2


def _fwd(self_s, hum, wr1, wrb1, wr2, wrb2, w1p, wb1p, w2p, wb2p,
         c1p, comb1, rel1, c1b, comb2, rel2, c2b,
         vw1, vb1, vw2, vb2, vw3, vb3, out):
    f32 = jnp.float32
    bf16 = jnp.bfloat16
    relu = jax.nn.relu

    def dot(a, b):
        return jnp.dot(a, b, preferred_element_type=f32,
                       precision=jax.lax.Precision.HIGHEST)

    r = relu(dot(relu(dot(self_s[:], wr1[:]) + wrb1[:]), wr2[:]) + wrb2[:])

    h1 = relu(dot(hum[:], w1p[:]) + wb1p[:])
    hf = relu(dot(h1, w2p[:]) + wb2p[:])

    hf3 = hf.reshape(_B, _P, 64)
    sall = hf3.sum(axis=1)
    s1 = sall[:, :32] + sall[:, 32:] - hf3[:, _P - 1, 32:] + r

    t1 = dot(s1, rel1[:]) + c1b[:]
    x1r = relu(dot(r, comb1[:]) + t1)
    t1p = jnp.concatenate([t1, t1], axis=1)
    x1 = relu((dot(hf, c1p[:])).reshape(_B, _P, 104)
              + t1p[:, None, :])

    zall = x1.sum(axis=1)
    s2 = zall[:, :52] + zall[:, 52:] - x1[:, _P - 1, 52:] + x1r

    x2 = relu(dot(x1r, comb2[:]) + dot(s2, rel2[:]) + c2b[:])

    v = relu(dot(x2, vw1[:]) + vb1[:])
    v = relu(dot(v, vw2[:]) + vb2[:])
    out[:] = dot(v, vw3[:]) + vb3[:]


def _blockdiag2(w):
    r, c = w.shape
    z = jnp.zeros((2 * r, 2 * c), w.dtype)
    return z.at[:r, :c].set(w).at[r:, c:].set(w)


def kernel(state, dropout, wr_w1, wr_b1, wr_w2, wr_b2, wh_w1, wh_b1, wh_w2,
           wh_b2, c1_rel_w, c1_rel_b, c1_root_w, c2_rel_w, c2_rel_b,
           c2_root_w, v_w1, v_b1, v_w2, v_b2, v_w3, v_b3):
    f32 = jnp.float32
    B, A, _ = state.shape

    self_s = state[:, 0, :6]
    hum = state[:, :, 6:]
    hum = jnp.pad(hum, ((0, 0), (0, _N - A), (0, 0)))
    hum = hum.reshape(B * _P, 14)

    row = lambda b: b.reshape(1, -1)
    comb1 = (c1_root_w - c1_rel_w).T
    args = (
        self_s, hum,
        wr_w1.T, row(wr_b1), wr_w2.T, row(wr_b2),
        _blockdiag2(wh_w1.T), jnp.tile(row(wh_b1), (1, 2)),
        _blockdiag2(wh_w2.T), jnp.tile(row(wh_b2), (1, 2)),
        _blockdiag2(comb1), comb1, c1_rel_w.T, row(c1_rel_b),
        (c2_root_w - c2_rel_w).T, c2_rel_w.T, row(c2_rel_b),
        v_w1.T, row(v_b1), v_w2.T, row(v_b2), v_w3.T, row(v_b3),
    )
    return pl.pallas_call(
        _fwd,
        out_shape=jax.ShapeDtypeStruct((B, 1), f32),
    )(*args)


# speedup vs baseline: 1.0917x; 1.0917x over previous
import jax
import jax.numpy as jnp
from jax.experimental import pallas as pl

_B = 64
_N = 128
_P = _N // 2


def _fwd(self_s, hum, wr1, wrb1, wr2, wrb2, w1p, wb1p, w2p, wb2p,
         c1p, comb1, rel1, c1b, comb2, rel2, c2b,
         vw1, vb1, vw2, vb2, vw3, vb3, out):
    f32 = jnp.float32
    bf16 = jnp.bfloat16
    relu = jax.nn.relu

    def dot(a, b):
        return jnp.dot(a, b, preferred_element_type=f32,
                       precision=jax.lax.Precision.HIGHEST)

    def dot4(a, b):
        ah = a.astype(bf16)
        al = (a - ah.astype(f32)).astype(bf16)
        bh = b.astype(bf16)
        bl = (b - bh.astype(f32)).astype(bf16)
        d = lambda x, y: jax.lax.dot_general(
            x, y, (((1,), (0,)), ((), ())), preferred_element_type=f32)
        return (d(ah, bh) + d(al, bh)) + (d(ah, bl) + d(al, bl))

    r = relu(dot(relu(dot(self_s[:], wr1[:]) + wrb1[:]), wr2[:]) + wrb2[:])

    h1 = relu(dot4(hum[:], w1p[:]) + wb1p[:])
    hf = relu(dot4(h1, w2p[:]) + wb2p[:])

    hf3 = hf.reshape(_B, _P, 64)
    sall = hf3.sum(axis=1)
    s1 = sall[:, :32] + sall[:, 32:] - hf3[:, _P - 1, 32:] + r

    t1 = dot(s1, rel1[:]) + c1b[:]
    x1r = relu(dot(r, comb1[:]) + t1)
    t1p = jnp.concatenate([t1, t1], axis=1)
    x1 = relu((dot4(hf, c1p[:])).reshape(_B, _P, 104)
              + t1p[:, None, :])

    zall = x1.sum(axis=1)
    s2 = zall[:, :52] + zall[:, 52:] - x1[:, _P - 1, 52:] + x1r

    x2 = relu(dot(x1r, comb2[:]) + dot(s2, rel2[:]) + c2b[:])

    v = relu(dot(x2, vw1[:]) + vb1[:])
    v = relu(dot(v, vw2[:]) + vb2[:])
    out[:] = dot(v, vw3[:]) + vb3[:]


def _blockdiag2(w):
    r, c = w.shape
    z = jnp.zeros((2 * r, 2 * c), w.dtype)
    return z.at[:r, :c].set(w).at[r:, c:].set(w)


def kernel(state, dropout, wr_w1, wr_b1, wr_w2, wr_b2, wh_w1, wh_b1, wh_w2,
           wh_b2, c1_rel_w, c1_rel_b, c1_root_w, c2_rel_w, c2_rel_b,
           c2_root_w, v_w1, v_b1, v_w2, v_b2, v_w3, v_b3):
    f32 = jnp.float32
    B, A, _ = state.shape

    self_s = state[:, 0, :6]
    hum = state[:, :, 6:]
    hum = jnp.pad(hum, ((0, 0), (0, _N - A), (0, 0)))
    hum = hum.reshape(B * _P, 14)

    row = lambda b: b.reshape(1, -1)
    comb1 = (c1_root_w - c1_rel_w).T
    args = (
        self_s, hum,
        wr_w1.T, row(wr_b1), wr_w2.T, row(wr_b2),
        _blockdiag2(wh_w1.T), jnp.tile(row(wh_b1), (1, 2)),
        _blockdiag2(wh_w2.T), jnp.tile(row(wh_b2), (1, 2)),
        _blockdiag2(comb1), comb1, c1_rel_w.T, row(c1_rel_b),
        (c2_root_w - c2_rel_w).T, c2_rel_w.T, row(c2_rel_b),
        v_w1.T, row(v_b1), v_w2.T, row(v_b2), v_w3.T, row(v_b3),
    )
    return pl.pallas_call(
        _fwd,
        out_shape=jax.ShapeDtypeStruct((B, 1), f32),
    )(*args)


# speedup vs baseline: 1.1920x; 1.0919x over previous
import jax
import jax.numpy as jnp
from jax.experimental import pallas as pl

_B = 64
_N = 128
_P = _N // 2


def _fwd(self_s, hum, wr1, wrb1, wr2, wrb2, wh1, whb1, wh2, whb2,
         comb1, rel1, c1b, comb2, rel2, c2b,
         vw1, vb1, vw2, vb2, vw3, vb3, out):
    f32 = jnp.float32
    bf16 = jnp.bfloat16
    relu = jax.nn.relu

    def dot(a, b):
        return jnp.dot(a, b, preferred_element_type=f32,
                       precision=jax.lax.Precision.HIGHEST)

    def dot4(a, b):
        ah = a.astype(bf16)
        al = (a - ah.astype(f32)).astype(bf16)
        bh = b.astype(bf16)
        bl = (b - bh.astype(f32)).astype(bf16)
        d = lambda x, y: jax.lax.dot_general(
            x, y, (((1,), (0,)), ((), ())), preferred_element_type=f32)
        return (d(ah, bh) + d(al, bh)) + (d(ah, bl) + d(al, bl))

    def blockdiag2(w):
        z = jnp.zeros(w.shape, w.dtype)
        return jnp.concatenate(
            [jnp.concatenate([w, z], axis=1),
             jnp.concatenate([z, w], axis=1)], axis=0)

    w1p = blockdiag2(wh1[:])
    wb1p = jnp.concatenate([whb1[:], whb1[:]], axis=1)
    w2p = blockdiag2(wh2[:])
    wb2p = jnp.concatenate([whb2[:], whb2[:]], axis=1)
    c1p = blockdiag2(comb1[:])

    r = relu(dot(relu(dot(self_s[:], wr1[:]) + wrb1[:]), wr2[:]) + wrb2[:])

    h1 = relu(dot4(hum[:], w1p) + wb1p)
    hf = relu(dot4(h1, w2p) + wb2p)

    hf3 = hf.reshape(_B, _P, 64)
    sall = hf3.sum(axis=1)
    s1 = sall[:, :32] + sall[:, 32:] - hf3[:, _P - 1, 32:] + r

    t1 = dot(s1, rel1[:]) + c1b[:]
    x1r = relu(dot(r, comb1[:]) + t1)
    t1p = jnp.concatenate([t1, t1], axis=1)
    x1 = relu((dot4(hf, c1p)).reshape(_B, _P, 104)
              + t1p[:, None, :])

    zall = x1.sum(axis=1)
    s2 = zall[:, :52] + zall[:, 52:] - x1[:, _P - 1, 52:] + x1r

    x2 = relu(dot(x1r, comb2[:]) + dot(s2, rel2[:]) + c2b[:])

    v = relu(dot(x2, vw1[:]) + vb1[:])
    v = relu(dot(v, vw2[:]) + vb2[:])
    out[:] = dot(v, vw3[:]) + vb3[:]


def kernel(state, dropout, wr_w1, wr_b1, wr_w2, wr_b2, wh_w1, wh_b1, wh_w2,
           wh_b2, c1_rel_w, c1_rel_b, c1_root_w, c2_rel_w, c2_rel_b,
           c2_root_w, v_w1, v_b1, v_w2, v_b2, v_w3, v_b3):
    f32 = jnp.float32
    B, A, _ = state.shape

    self_s = state[:, 0, :6]
    hum = state[:, :, 6:]
    hum = jnp.pad(hum, ((0, 0), (0, _N - A), (0, 0)))
    hum = hum.reshape(B * _P, 14)

    row = lambda b: b.reshape(1, -1)
    args = (
        self_s, hum,
        wr_w1.T, row(wr_b1), wr_w2.T, row(wr_b2),
        wh_w1.T, row(wh_b1), wh_w2.T, row(wh_b2),
        (c1_root_w - c1_rel_w).T, c1_rel_w.T, row(c1_rel_b),
        (c2_root_w - c2_rel_w).T, c2_rel_w.T, row(c2_rel_b),
        v_w1.T, row(v_b1), v_w2.T, row(v_b2), v_w3.T, row(v_b3),
    )
    return pl.pallas_call(
        _fwd,
        out_shape=jax.ShapeDtypeStruct((B, 1), f32),
    )(*args)
